# Initial kernel scaffold; baseline (speedup 1.0000x reference)
import jax
import jax.numpy as jnp
from jax.experimental import pallas as pl


def kernel(x, edge_index, W_enc, b_enc, W_tgt, b_tgt, W1, b1, prelu_a, W2, b2):
    raise NotImplementedError("write your pallas kernel here")


# speedup vs baseline: 8.1436x; 8.1436x over previous
import functools

import jax
import jax.numpy as jnp
from jax import lax
from jax.experimental import pallas as pl
from jax.experimental.pallas import tpu as pltpu
from jax.experimental.pallas import tpu_sc as plsc

N = 10000
E = 320000
D = 128
NHOP = 10

NC = 2
NS = 16
HD = D // NC
NPAD = 10240
RPT = NPAD // NS
NV = RPT // 16
CHUNK = 128
SUP = 16
NSUP = 10
EPAD = NS * NSUP * SUP * CHUNK
QC = RPT // CHUNK

_mesh = plsc.VectorSubcoreMesh(
    core_axis_name="c", subcore_axis_name="s", num_cores=NC, num_subcores=NS)
_sc_params = pltpu.CompilerParams(
    needs_layout_passes=False, use_tc_tiling_on_sc=False)


def _rsqrt16(x):
  i = plsc.bitcast(x, jnp.int32)
  i = jnp.int32(0x5F3759DF) - (i >> 1)
  y = plsc.bitcast(i, jnp.float32)
  for _ in range(3):
    y = y * (1.5 - 0.5 * x * y * y)
  return y


def _zero_rows(buf, nrows):
  z = jnp.zeros((16,), jnp.float32)
  def row(r, _):
    for j in range(HD // 16):
      buf[r, pl.ds(j * 16, 16)] = z
    return 0
  lax.fori_loop(0, nrows, row, 0)


def _fill_flat(buf, nvec, value):
  v = jnp.full((16,), value, jnp.float32)
  def vec(i, _):
    buf[pl.ds(i * 16, 16)] = v
    return 0
  lax.fori_loop(0, nvec, vec, 0)


def _scale_block(buf, pref, base):
  def row(r, _):
    v = plsc.load_gather(pref, [jnp.full((16,), r, jnp.int32) + base])
    for j in range(HD // 16):
      buf[r, pl.ds(j * 16, 16)] = buf[r, pl.ds(j * 16, 16)] * v
    return 0
  lax.fori_loop(0, CHUNK, row, 0)


def _edge_pass(A, B, srcg, dstg, s, srcsb, dstsb, grow, sem):
  def sup(u, _):
    pltpu.sync_copy(srcg.at[s].at[u], srcsb)
    pltpu.sync_copy(dstg.at[s].at[u], dstsb)
    def chunk(i, _):
      pltpu.async_copy(A.at[srcsb.at[i]], grow, sem).wait()
      pltpu.sync_copy(grow, B.at[dstsb.at[i]], add=True)
      return 0
    lax.fori_loop(0, SUP, chunk, 0)
    return 0
  lax.fori_loop(0, NSUP, sup, 0)


def _rsqrt_slice(deg_sh, tmpf, dest, r0):
  pltpu.sync_copy(deg_sh.at[pl.ds(r0, RPT)], tmpf)
  def vec(v, _):
    sl = pl.ds(v * 16, 16)
    dest[sl] = _rsqrt16(jnp.maximum(tmpf[sl], 1.0))
    return 0
  lax.fori_loop(0, NV, vec, 0)


def _sc_stage1_body(xh, srcg, dstg, aggh, rin_o, rout_o, cmid_o,
                    A, B, dgo_sh, dgi_sh, srcsb, dstsb, sbuf, grow, zb,
                    onesb, routp, rinp, tmpf, accb, sem):
  c = lax.axis_index("c")
  s = lax.axis_index("s")
  r0 = s * RPT

  _zero_rows(zb, CHUNK)
  _fill_flat(onesb, CHUNK // 16, 1.0)
  _fill_flat(tmpf, NV, 0.0)
  pltpu.sync_copy(tmpf, dgo_sh.at[pl.ds(r0, RPT)])
  pltpu.sync_copy(tmpf, dgi_sh.at[pl.ds(r0, RPT)])
  plsc.subcore_barrier()

  def sup(u, _):
    pltpu.sync_copy(srcg.at[s].at[u], srcsb)
    pltpu.sync_copy(dstg.at[s].at[u], dstsb)
    def chunk(i, _):
      pltpu.sync_copy(onesb, dgo_sh.at[srcsb.at[i]], add=True)
      pltpu.sync_copy(onesb, dgi_sh.at[dstsb.at[i]], add=True)
      return 0
    lax.fori_loop(0, SUP, chunk, 0)
    return 0
  lax.fori_loop(0, NSUP, sup, 0)
  plsc.subcore_barrier()

  _rsqrt_slice(dgo_sh, tmpf, routp, r0)
  _rsqrt_slice(dgi_sh, tmpf, rinp, r0)

  @pl.when(c == 0)
  def _():
    pltpu.sync_copy(routp, rout_o.at[pl.ds(r0, RPT)])
    pltpu.sync_copy(rinp, rin_o.at[pl.ds(r0, RPT)])
    def cv(v, _):
      sl = pl.ds(v * 16, 16)
      accb[sl] = rinp[sl] * routp[sl]
      return 0
    lax.fori_loop(0, NV, cv, 0)
    pltpu.sync_copy(accb, cmid_o.at[pl.ds(r0, RPT)])

  for q in range(QC):
    blk = pl.ds(r0 + q * CHUNK, CHUNK)
    pltpu.sync_copy(xh.at[c].at[blk], sbuf)
    _scale_block(sbuf, routp, q * CHUNK)
    pltpu.sync_copy(sbuf, A.at[blk])
    pltpu.sync_copy(zb, B.at[blk])
  plsc.subcore_barrier()

  _edge_pass(A, B, srcg, dstg, s, srcsb, dstsb, grow, sem)
  plsc.subcore_barrier()
  for q in range(QC):
    blk = pl.ds(r0 + q * CHUNK, CHUNK)
    pltpu.sync_copy(B.at[blk], sbuf)
    _scale_block(sbuf, rinp, q * CHUNK)
    pltpu.sync_copy(sbuf, aggh.at[c].at[blk])


def _sc_hops_body(h1h, srcg, dstg, rin_i, rout_i, cmid_i, hh,
                  A, B, srcsb, dstsb, sbuf, grow, zb,
                  routp, rinp, cmp_, sem):
  c = lax.axis_index("c")
  s = lax.axis_index("s")
  r0 = s * RPT

  _zero_rows(zb, CHUNK)
  pltpu.sync_copy(rout_i.at[pl.ds(r0, RPT)], routp)
  pltpu.sync_copy(rin_i.at[pl.ds(r0, RPT)], rinp)
  pltpu.sync_copy(cmid_i.at[pl.ds(r0, RPT)], cmp_)

  for q in range(QC):
    blk = pl.ds(r0 + q * CHUNK, CHUNK)
    pltpu.sync_copy(h1h.at[c].at[blk], sbuf)
    _scale_block(sbuf, routp, q * CHUNK)
    pltpu.sync_copy(sbuf, A.at[blk])
    pltpu.sync_copy(zb, B.at[blk])
  plsc.subcore_barrier()

  for k in range(NHOP):
    _edge_pass(A, B, srcg, dstg, s, srcsb, dstsb, grow, sem)
    plsc.subcore_barrier()
    if k < NHOP - 1:
      for q in range(QC):
        blk = pl.ds(r0 + q * CHUNK, CHUNK)
        pltpu.sync_copy(B.at[blk], sbuf)
        _scale_block(sbuf, cmp_, q * CHUNK)
        pltpu.sync_copy(sbuf, A.at[blk])
        pltpu.sync_copy(zb, B.at[blk])
      plsc.subcore_barrier()
    else:
      for q in range(QC):
        blk = pl.ds(r0 + q * CHUNK, CHUNK)
        pltpu.sync_copy(B.at[blk], sbuf)
        _scale_block(sbuf, rinp, q * CHUNK)
        pltpu.sync_copy(h1h.at[c].at[blk], grow)
        def addr(r, _):
          for j in range(HD // 16):
            sl = pl.ds(j * 16, 16)
            sbuf[r, sl] = sbuf[r, sl] + grow[r, sl]
          return 0
        lax.fori_loop(0, CHUNK, addr, 0)
        pltpu.sync_copy(sbuf, hh.at[c].at[blk])


_sc_stage1 = functools.partial(
    pl.kernel,
    compiler_params=_sc_params,
    out_type=(
        jax.ShapeDtypeStruct((NC, NPAD, HD), jnp.float32),
        jax.ShapeDtypeStruct((NPAD,), jnp.float32),
        jax.ShapeDtypeStruct((NPAD,), jnp.float32),
        jax.ShapeDtypeStruct((NPAD,), jnp.float32),
    ),
    mesh=_mesh,
    scratch_types=[
        pltpu.VMEM_SHARED((NPAD, HD), jnp.float32),
        pltpu.VMEM_SHARED((NPAD, HD), jnp.float32),
        pltpu.VMEM_SHARED((NPAD,), jnp.float32),
        pltpu.VMEM_SHARED((NPAD,), jnp.float32),
        pltpu.VMEM((SUP, CHUNK), jnp.int32),
        pltpu.VMEM((SUP, CHUNK), jnp.int32),
        pltpu.VMEM((CHUNK, HD), jnp.float32),
        pltpu.VMEM((CHUNK, HD), jnp.float32),
        pltpu.VMEM((CHUNK, HD), jnp.float32),
        pltpu.VMEM((CHUNK,), jnp.float32),
        pltpu.VMEM((RPT,), jnp.float32),
        pltpu.VMEM((RPT,), jnp.float32),
        pltpu.VMEM((RPT,), jnp.float32),
        pltpu.VMEM((RPT,), jnp.float32),
        pltpu.SemaphoreType.DMA,
    ])(_sc_stage1_body)

_sc_hops = functools.partial(
    pl.kernel,
    compiler_params=_sc_params,
    out_type=jax.ShapeDtypeStruct((NC, NPAD, HD), jnp.float32),
    mesh=_mesh,
    scratch_types=[
        pltpu.VMEM_SHARED((NPAD, HD), jnp.float32),
        pltpu.VMEM_SHARED((NPAD, HD), jnp.float32),
        pltpu.VMEM((SUP, CHUNK), jnp.int32),
        pltpu.VMEM((SUP, CHUNK), jnp.int32),
        pltpu.VMEM((CHUNK, HD), jnp.float32),
        pltpu.VMEM((CHUNK, HD), jnp.float32),
        pltpu.VMEM((CHUNK, HD), jnp.float32),
        pltpu.VMEM((RPT,), jnp.float32),
        pltpu.VMEM((RPT,), jnp.float32),
        pltpu.VMEM((RPT,), jnp.float32),
        pltpu.SemaphoreType.DMA,
    ])(_sc_hops_body)


BLK = 1024


def _tc_enc_body(aL, aR, We, be, Wt, bt, h1h, ht):
  lo, hi = pl.ds(0, HD), pl.ds(HD, HD)
  pre = aL[...] @ We[lo, :] + aR[...] @ We[hi, :] + be[...]
  h1 = jnp.maximum(pre, 0.0)
  h1h[0, :, :] = h1[:, :HD]
  h1h[1, :, :] = h1[:, HD:]
  pre_t = aL[...] @ Wt[lo, :] + aR[...] @ Wt[hi, :] + bt[...]
  ht[...] = jnp.maximum(pre_t, 0.0)


def _tc_pred_body(hL, hR, W1, b1, ap, W2, b2, out):
  lo, hi = pl.ds(0, HD), pl.ds(HD, HD)
  t = hL[...] @ W1[lo, :] + hR[...] @ W1[hi, :] + b1[...]
  t = jnp.where(t >= 0.0, t, ap[...] * t)
  out[...] = t @ W2[...] + b2[...]


def _full(shape):
  return pl.BlockSpec(shape, lambda i: tuple(0 for _ in shape))


def _tc_enc(aggh, W_enc, b_enc, W_tgt, b_tgt):
  return pl.pallas_call(
      _tc_enc_body,
      grid=(NPAD // BLK,),
      in_specs=[
          pl.BlockSpec((BLK, HD), lambda i: (i, 0)),
          pl.BlockSpec((BLK, HD), lambda i: (i, 0)),
          _full((D, D)), _full((1, D)), _full((D, D)), _full((1, D)),
      ],
      out_specs=[
          pl.BlockSpec((NC, BLK, HD), lambda i: (0, i, 0)),
          pl.BlockSpec((BLK, D), lambda i: (i, 0)),
      ],
      out_shape=[
          jax.ShapeDtypeStruct((NC, NPAD, HD), jnp.float32),
          jax.ShapeDtypeStruct((NPAD, D), jnp.float32),
      ],
  )(aggh[0], aggh[1], W_enc, b_enc.reshape(1, D), W_tgt, b_tgt.reshape(1, D))


def _tc_pred(hh, W1, b1, prelu_a, W2, b2):
  return pl.pallas_call(
      _tc_pred_body,
      grid=(NPAD // BLK,),
      in_specs=[
          pl.BlockSpec((BLK, HD), lambda i: (i, 0)),
          pl.BlockSpec((BLK, HD), lambda i: (i, 0)),
          _full((D, D)), _full((1, D)), _full((1, 1)), _full((D, D)),
          _full((1, D)),
      ],
      out_specs=pl.BlockSpec((BLK, D), lambda i: (i, 0)),
      out_shape=jax.ShapeDtypeStruct((NPAD, D), jnp.float32),
  )(hh[0], hh[1], W1, b1.reshape(1, D), prelu_a.reshape(1, 1), W2,
    b2.reshape(1, D))


def kernel(x, edge_index, W_enc, b_enc, W_tgt, b_tgt, W1, b1, prelu_a, W2, b2):
  src = edge_index[0]
  dst = edge_index[1]
  srcg = jnp.pad(src, (0, EPAD - E), constant_values=N).reshape(
      NS, NSUP, SUP, CHUNK)
  dstg = jnp.pad(dst, (0, EPAD - E), constant_values=N).reshape(
      NS, NSUP, SUP, CHUNK)
  xh = jnp.pad(x, ((0, NPAD - N), (0, 0))).reshape(NPAD, NC, HD).transpose(
      1, 0, 2)

  aggh, rin, rout, cmid = _sc_stage1(xh, srcg, dstg)
  h1h, h_target = _tc_enc(aggh, W_enc, b_enc, W_tgt, b_tgt)
  hh = _sc_hops(h1h, srcg, dstg, rin, rout, cmid)
  h_pred = _tc_pred(hh, W1, b1, prelu_a, W2, b2)

  h = jnp.concatenate([hh[0, :N], hh[1, :N]], axis=1)
  return h, h_pred[:N], h_target[:N]


# speedup vs baseline: 10.2395x; 1.2574x over previous
import functools

import jax
import jax.numpy as jnp
from jax import lax
from jax.experimental import pallas as pl
from jax.experimental.pallas import tpu as pltpu
from jax.experimental.pallas import tpu_sc as plsc

N = 10000
E = 320000
D = 128
NHOP = 10

NC = 2
NS = 16
HD = D // NC
NPAD = 10240
RPT = NPAD // NS
NV = RPT // 16
CHUNK = 128
SUP = 16
NSUP = 10
EPAD = NS * NSUP * SUP * CHUNK
QC = RPT // CHUNK

_mesh = plsc.VectorSubcoreMesh(
    core_axis_name="c", subcore_axis_name="s", num_cores=NC, num_subcores=NS)
_sc_params = pltpu.CompilerParams(
    needs_layout_passes=False, use_tc_tiling_on_sc=False)


def _rsqrt16(x):
  i = plsc.bitcast(x, jnp.int32)
  i = jnp.int32(0x5F3759DF) - (i >> 1)
  y = plsc.bitcast(i, jnp.float32)
  for _ in range(3):
    y = y * (1.5 - 0.5 * x * y * y)
  return y


def _zero_rows(buf, nrows):
  z = jnp.zeros((16,), jnp.float32)
  def row(r, _):
    for j in range(HD // 16):
      buf[r, pl.ds(j * 16, 16)] = z
    return 0
  lax.fori_loop(0, nrows, row, 0)


def _fill_flat(buf, nvec, value):
  v = jnp.full((16,), value, jnp.float32)
  def vec(i, _):
    buf[pl.ds(i * 16, 16)] = v
    return 0
  lax.fori_loop(0, nvec, vec, 0)


def _scale_block(buf, pref, base):
  def row(r, _):
    v = plsc.load_gather(pref, [jnp.full((16,), r, jnp.int32) + base])
    for j in range(HD // 16):
      buf[r, pl.ds(j * 16, 16)] = buf[r, pl.ds(j * 16, 16)] * v
    return 0
  lax.fori_loop(0, CHUNK, row, 0)


def _edge_pass(A, B, srcg, dstg, s, srcsb, dstsb, g0, g1, gs0, gs1, ss0, ss1):
  g = (g0, g1)
  gsem = (gs0, gs1)
  ssem = (ss0, ss1)
  def sup(u, _):
    pltpu.sync_copy(srcg.at[s].at[u], srcsb)
    pltpu.sync_copy(dstg.at[s].at[u], dstsb)
    sd = [None, None]
    gd = pltpu.async_copy(A.at[srcsb.at[0]], g[0], gsem[0])
    for i in range(SUP):
      p = i % 2
      q = (i + 1) % 2
      if i < SUP - 1:
        if sd[q] is not None:
          sd[q].wait()
        gd_next = pltpu.async_copy(A.at[srcsb.at[i + 1]], g[q], gsem[q])
      gd.wait()
      sd[p] = pltpu.async_copy(g[p], B.at[dstsb.at[i]], ssem[p], add=True)
      if i < SUP - 1:
        gd = gd_next
    sd[0].wait()
    sd[1].wait()
    return 0
  lax.fori_loop(0, NSUP, sup, 0)


def _rsqrt_slice(deg_sh, tmpf, dest, r0):
  pltpu.sync_copy(deg_sh.at[pl.ds(r0, RPT)], tmpf)
  def vec(v, _):
    sl = pl.ds(v * 16, 16)
    dest[sl] = _rsqrt16(jnp.maximum(tmpf[sl], 1.0))
    return 0
  lax.fori_loop(0, NV, vec, 0)


def _sc_stage1_body(xh, srcg, dstg, aggh, rin_o, rout_o, cmid_o,
                    A, B, dgo_sh, dgi_sh, srcsb, dstsb, sbuf, g0, g1, zb,
                    onesb, routp, rinp, tmpf, accb, gs0, gs1, ss0, ss1):
  c = lax.axis_index("c")
  s = lax.axis_index("s")
  r0 = s * RPT

  _zero_rows(zb, CHUNK)
  _fill_flat(onesb, CHUNK // 16, 1.0)
  _fill_flat(tmpf, NV, 0.0)
  pltpu.sync_copy(tmpf, dgo_sh.at[pl.ds(r0, RPT)])
  pltpu.sync_copy(tmpf, dgi_sh.at[pl.ds(r0, RPT)])
  plsc.subcore_barrier()

  def sup(u, _):
    pltpu.sync_copy(srcg.at[s].at[u], srcsb)
    pltpu.sync_copy(dstg.at[s].at[u], dstsb)
    def chunk(i, _):
      pltpu.sync_copy(onesb, dgo_sh.at[srcsb.at[i]], add=True)
      pltpu.sync_copy(onesb, dgi_sh.at[dstsb.at[i]], add=True)
      return 0
    lax.fori_loop(0, SUP, chunk, 0)
    return 0
  lax.fori_loop(0, NSUP, sup, 0)
  plsc.subcore_barrier()

  _rsqrt_slice(dgo_sh, tmpf, routp, r0)
  _rsqrt_slice(dgi_sh, tmpf, rinp, r0)

  @pl.when(c == 0)
  def _():
    pltpu.sync_copy(routp, rout_o.at[pl.ds(r0, RPT)])
    pltpu.sync_copy(rinp, rin_o.at[pl.ds(r0, RPT)])
    def cv(v, _):
      sl = pl.ds(v * 16, 16)
      accb[sl] = rinp[sl] * routp[sl]
      return 0
    lax.fori_loop(0, NV, cv, 0)
    pltpu.sync_copy(accb, cmid_o.at[pl.ds(r0, RPT)])

  for q in range(QC):
    blk = pl.ds(r0 + q * CHUNK, CHUNK)
    pltpu.sync_copy(xh.at[c].at[blk], sbuf)
    _scale_block(sbuf, routp, q * CHUNK)
    pltpu.sync_copy(sbuf, A.at[blk])
    pltpu.sync_copy(zb, B.at[blk])
  plsc.subcore_barrier()

  _edge_pass(A, B, srcg, dstg, s, srcsb, dstsb, g0, g1, gs0, gs1, ss0, ss1)
  plsc.subcore_barrier()
  for q in range(QC):
    blk = pl.ds(r0 + q * CHUNK, CHUNK)
    pltpu.sync_copy(B.at[blk], sbuf)
    _scale_block(sbuf, rinp, q * CHUNK)
    pltpu.sync_copy(sbuf, aggh.at[c].at[blk])


def _sc_hops_body(h1h, srcg, dstg, rin_i, rout_i, cmid_i, hh,
                  A, B, srcsb, dstsb, sbuf, g0, g1, zb,
                  routp, rinp, cmp_, gs0, gs1, ss0, ss1):
  c = lax.axis_index("c")
  s = lax.axis_index("s")
  r0 = s * RPT

  _zero_rows(zb, CHUNK)
  pltpu.sync_copy(rout_i.at[pl.ds(r0, RPT)], routp)
  pltpu.sync_copy(rin_i.at[pl.ds(r0, RPT)], rinp)
  pltpu.sync_copy(cmid_i.at[pl.ds(r0, RPT)], cmp_)

  for q in range(QC):
    blk = pl.ds(r0 + q * CHUNK, CHUNK)
    pltpu.sync_copy(h1h.at[c].at[blk], sbuf)
    _scale_block(sbuf, routp, q * CHUNK)
    pltpu.sync_copy(sbuf, A.at[blk])
    pltpu.sync_copy(zb, B.at[blk])
  plsc.subcore_barrier()

  for k in range(NHOP):
    _edge_pass(A, B, srcg, dstg, s, srcsb, dstsb, g0, g1, gs0, gs1, ss0, ss1)
    plsc.subcore_barrier()
    if k < NHOP - 1:
      for q in range(QC):
        blk = pl.ds(r0 + q * CHUNK, CHUNK)
        pltpu.sync_copy(B.at[blk], sbuf)
        _scale_block(sbuf, cmp_, q * CHUNK)
        pltpu.sync_copy(sbuf, A.at[blk])
        pltpu.sync_copy(zb, B.at[blk])
      plsc.subcore_barrier()
    else:
      for q in range(QC):
        blk = pl.ds(r0 + q * CHUNK, CHUNK)
        pltpu.sync_copy(B.at[blk], sbuf)
        _scale_block(sbuf, rinp, q * CHUNK)
        pltpu.sync_copy(h1h.at[c].at[blk], g0)
        def addr(r, _):
          for j in range(HD // 16):
            sl = pl.ds(j * 16, 16)
            sbuf[r, sl] = sbuf[r, sl] + g0[r, sl]
          return 0
        lax.fori_loop(0, CHUNK, addr, 0)
        pltpu.sync_copy(sbuf, hh.at[c].at[blk])


_sc_stage1 = functools.partial(
    pl.kernel,
    compiler_params=_sc_params,
    out_type=(
        jax.ShapeDtypeStruct((NC, NPAD, HD), jnp.float32),
        jax.ShapeDtypeStruct((NPAD,), jnp.float32),
        jax.ShapeDtypeStruct((NPAD,), jnp.float32),
        jax.ShapeDtypeStruct((NPAD,), jnp.float32),
    ),
    mesh=_mesh,
    scratch_types=[
        pltpu.VMEM_SHARED((NPAD, HD), jnp.float32),
        pltpu.VMEM_SHARED((NPAD, HD), jnp.float32),
        pltpu.VMEM_SHARED((NPAD,), jnp.float32),
        pltpu.VMEM_SHARED((NPAD,), jnp.float32),
        pltpu.VMEM((SUP, CHUNK), jnp.int32),
        pltpu.VMEM((SUP, CHUNK), jnp.int32),
        pltpu.VMEM((CHUNK, HD), jnp.float32),
        pltpu.VMEM((CHUNK, HD), jnp.float32),
        pltpu.VMEM((CHUNK, HD), jnp.float32),
        pltpu.VMEM((CHUNK, HD), jnp.float32),
        pltpu.VMEM((CHUNK,), jnp.float32),
        pltpu.VMEM((RPT,), jnp.float32),
        pltpu.VMEM((RPT,), jnp.float32),
        pltpu.VMEM((RPT,), jnp.float32),
        pltpu.VMEM((RPT,), jnp.float32),
        pltpu.SemaphoreType.DMA,
        pltpu.SemaphoreType.DMA,
        pltpu.SemaphoreType.DMA,
        pltpu.SemaphoreType.DMA,
    ])(_sc_stage1_body)

_sc_hops = functools.partial(
    pl.kernel,
    compiler_params=_sc_params,
    out_type=jax.ShapeDtypeStruct((NC, NPAD, HD), jnp.float32),
    mesh=_mesh,
    scratch_types=[
        pltpu.VMEM_SHARED((NPAD, HD), jnp.float32),
        pltpu.VMEM_SHARED((NPAD, HD), jnp.float32),
        pltpu.VMEM((SUP, CHUNK), jnp.int32),
        pltpu.VMEM((SUP, CHUNK), jnp.int32),
        pltpu.VMEM((CHUNK, HD), jnp.float32),
        pltpu.VMEM((CHUNK, HD), jnp.float32),
        pltpu.VMEM((CHUNK, HD), jnp.float32),
        pltpu.VMEM((CHUNK, HD), jnp.float32),
        pltpu.VMEM((RPT,), jnp.float32),
        pltpu.VMEM((RPT,), jnp.float32),
        pltpu.VMEM((RPT,), jnp.float32),
        pltpu.SemaphoreType.DMA,
        pltpu.SemaphoreType.DMA,
        pltpu.SemaphoreType.DMA,
        pltpu.SemaphoreType.DMA,
    ])(_sc_hops_body)


BLK = 1024


def _tc_enc_body(aL, aR, We, be, Wt, bt, h1h, ht):
  lo, hi = pl.ds(0, HD), pl.ds(HD, HD)
  pre = aL[...] @ We[lo, :] + aR[...] @ We[hi, :] + be[...]
  h1 = jnp.maximum(pre, 0.0)
  h1h[0, :, :] = h1[:, :HD]
  h1h[1, :, :] = h1[:, HD:]
  pre_t = aL[...] @ Wt[lo, :] + aR[...] @ Wt[hi, :] + bt[...]
  ht[...] = jnp.maximum(pre_t, 0.0)


def _tc_pred_body(hL, hR, W1, b1, ap, W2, b2, out):
  lo, hi = pl.ds(0, HD), pl.ds(HD, HD)
  t = hL[...] @ W1[lo, :] + hR[...] @ W1[hi, :] + b1[...]
  t = jnp.where(t >= 0.0, t, ap[...] * t)
  out[...] = t @ W2[...] + b2[...]


def _full(shape):
  return pl.BlockSpec(shape, lambda i: tuple(0 for _ in shape))


def _tc_enc(aggh, W_enc, b_enc, W_tgt, b_tgt):
  return pl.pallas_call(
      _tc_enc_body,
      grid=(NPAD // BLK,),
      in_specs=[
          pl.BlockSpec((BLK, HD), lambda i: (i, 0)),
          pl.BlockSpec((BLK, HD), lambda i: (i, 0)),
          _full((D, D)), _full((1, D)), _full((D, D)), _full((1, D)),
      ],
      out_specs=[
          pl.BlockSpec((NC, BLK, HD), lambda i: (0, i, 0)),
          pl.BlockSpec((BLK, D), lambda i: (i, 0)),
      ],
      out_shape=[
          jax.ShapeDtypeStruct((NC, NPAD, HD), jnp.float32),
          jax.ShapeDtypeStruct((NPAD, D), jnp.float32),
      ],
  )(aggh[0], aggh[1], W_enc, b_enc.reshape(1, D), W_tgt, b_tgt.reshape(1, D))


def _tc_pred(hh, W1, b1, prelu_a, W2, b2):
  return pl.pallas_call(
      _tc_pred_body,
      grid=(NPAD // BLK,),
      in_specs=[
          pl.BlockSpec((BLK, HD), lambda i: (i, 0)),
          pl.BlockSpec((BLK, HD), lambda i: (i, 0)),
          _full((D, D)), _full((1, D)), _full((1, 1)), _full((D, D)),
          _full((1, D)),
      ],
      out_specs=pl.BlockSpec((BLK, D), lambda i: (i, 0)),
      out_shape=jax.ShapeDtypeStruct((NPAD, D), jnp.float32),
  )(hh[0], hh[1], W1, b1.reshape(1, D), prelu_a.reshape(1, 1), W2,
    b2.reshape(1, D))


def kernel(x, edge_index, W_enc, b_enc, W_tgt, b_tgt, W1, b1, prelu_a, W2, b2):
  src = edge_index[0]
  dst = edge_index[1]
  srcg = jnp.pad(src, (0, EPAD - E), constant_values=N).reshape(
      NS, NSUP, SUP, CHUNK)
  dstg = jnp.pad(dst, (0, EPAD - E), constant_values=N).reshape(
      NS, NSUP, SUP, CHUNK)
  xh = jnp.pad(x, ((0, NPAD - N), (0, 0))).reshape(NPAD, NC, HD).transpose(
      1, 0, 2)

  aggh, rin, rout, cmid = _sc_stage1(xh, srcg, dstg)
  h1h, h_target = _tc_enc(aggh, W_enc, b_enc, W_tgt, b_tgt)
  hh = _sc_hops(h1h, srcg, dstg, rin, rout, cmid)
  h_pred = _tc_pred(hh, W1, b1, prelu_a, W2, b2)

  h = jnp.concatenate([hh[0, :N], hh[1, :N]], axis=1)
  return h, h_pred[:N], h_target[:N]


# speedup vs baseline: 12.8577x; 1.2557x over previous
import functools

import jax
import jax.numpy as jnp
from jax import lax
from jax.experimental import pallas as pl
from jax.experimental.pallas import tpu as pltpu
from jax.experimental.pallas import tpu_sc as plsc

N = 10000
E = 320000
D = 128
NHOP = 10

NC = 2
NS = 16
HD = D // NC
NPAD = 10240
RPT = NPAD // NS
NV = RPT // 16
CHUNK = 128
SUP = 16
NSUP = 10
EPAD = NS * NSUP * SUP * CHUNK
QC = RPT // CHUNK
NB = 4
LOOK = 3

_mesh = plsc.VectorSubcoreMesh(
    core_axis_name="c", subcore_axis_name="s", num_cores=NC, num_subcores=NS)
_sc_params = pltpu.CompilerParams(
    needs_layout_passes=False, use_tc_tiling_on_sc=False)


def _rsqrt16(x):
  i = plsc.bitcast(x, jnp.int32)
  i = jnp.int32(0x5F3759DF) - (i >> 1)
  y = plsc.bitcast(i, jnp.float32)
  for _ in range(3):
    y = y * (1.5 - 0.5 * x * y * y)
  return y


def _zero_rows(buf, nrows):
  z = jnp.zeros((16,), jnp.float32)
  def row(r, _):
    for j in range(HD // 16):
      buf[r, pl.ds(j * 16, 16)] = z
    return 0
  lax.fori_loop(0, nrows, row, 0)


def _fill_flat(buf, nvec, value):
  v = jnp.full((16,), value, jnp.float32)
  def vec(i, _):
    buf[pl.ds(i * 16, 16)] = v
    return 0
  lax.fori_loop(0, nvec, vec, 0)


def _scale_block(buf, pref, base):
  def row(r, _):
    v = plsc.load_gather(pref, [jnp.full((16,), r, jnp.int32) + base])
    for j in range(HD // 16):
      buf[r, pl.ds(j * 16, 16)] = buf[r, pl.ds(j * 16, 16)] * v
    return 0
  lax.fori_loop(0, CHUNK, row, 0)


def _edge_pass(A, B, srcg, dstg, s, srcsb, dstsb, gbufs, gsems, ssems,
               isem_s, isem_d):
  sd = [None] * NB
  gd = [None] * NB
  idw = [None, None]
  idw[0] = (pltpu.async_copy(srcg.at[s].at[0], srcsb.at[0], isem_s),
            pltpu.async_copy(dstg.at[s].at[0], dstsb.at[0], isem_d))
  for u in range(NSUP):
    par = u % 2
    idw[par][0].wait()
    idw[par][1].wait()
    if u < NSUP - 1:
      npar = (u + 1) % 2
      idw[npar] = (
          pltpu.async_copy(srcg.at[s].at[u + 1], srcsb.at[npar], isem_s),
          pltpu.async_copy(dstg.at[s].at[u + 1], dstsb.at[npar], isem_d))
    S = srcsb.at[par]
    Dx = dstsb.at[par]
    for i in range(SUP):
      p = i % NB
      if sd[p] is not None:
        sd[p].wait()
        sd[p] = None
      gd[p] = pltpu.async_copy(A.at[S.at[i]], gbufs[p], gsems[p])
      if i >= LOOK - 1:
        t = i - LOOK + 1
        pt = t % NB
        gd[pt].wait()
        sd[pt] = pltpu.async_copy(gbufs[pt], B.at[Dx.at[t]], ssems[pt],
                                  add=True)
    for t in range(SUP - LOOK + 1, SUP):
      pt = t % NB
      gd[pt].wait()
      sd[pt] = pltpu.async_copy(gbufs[pt], B.at[Dx.at[t]], ssems[pt],
                                add=True)
    for p in range(NB):
      if sd[p] is not None:
        sd[p].wait()
        sd[p] = None


def _rsqrt_slice(deg_sh, tmpf, dest, r0):
  pltpu.sync_copy(deg_sh.at[pl.ds(r0, RPT)], tmpf)
  def vec(v, _):
    sl = pl.ds(v * 16, 16)
    dest[sl] = _rsqrt16(jnp.maximum(tmpf[sl], 1.0))
    return 0
  lax.fori_loop(0, NV, vec, 0)


def _sc_stage1_body(xh, srcg, dstg, aggh, rin_o, rout_o, cmid_o,
                    A, B, dgo_sh, dgi_sh, srcsb, dstsb, g0, g1, g2, g3,
                    onesb, routp, rinp, tmpf, accb,
                    gs0, gs1, gs2, gs3, ss0, ss1, ss2, ss3, is_s, is_d):
  c = lax.axis_index("c")
  s = lax.axis_index("s")
  r0 = s * RPT
  gbufs = (g0, g1, g2, g3)
  gsems = (gs0, gs1, gs2, gs3)
  ssems = (ss0, ss1, ss2, ss3)

  _fill_flat(onesb, CHUNK // 16, 1.0)
  _fill_flat(tmpf, NV, 0.0)
  pltpu.sync_copy(tmpf, dgo_sh.at[pl.ds(r0, RPT)])
  pltpu.sync_copy(tmpf, dgi_sh.at[pl.ds(r0, RPT)])
  plsc.subcore_barrier()

  def sup(u, _):
    pltpu.sync_copy(srcg.at[s].at[u], srcsb.at[0])
    pltpu.sync_copy(dstg.at[s].at[u], dstsb.at[0])
    def chunk(i, _):
      pltpu.sync_copy(onesb, dgo_sh.at[srcsb.at[0].at[i]], add=True)
      pltpu.sync_copy(onesb, dgi_sh.at[dstsb.at[0].at[i]], add=True)
      return 0
    lax.fori_loop(0, SUP, chunk, 0)
    return 0
  lax.fori_loop(0, NSUP, sup, 0)
  plsc.subcore_barrier()

  _rsqrt_slice(dgo_sh, tmpf, routp, r0)
  _rsqrt_slice(dgi_sh, tmpf, rinp, r0)

  @pl.when(c == 0)
  def _():
    pltpu.sync_copy(routp, rout_o.at[pl.ds(r0, RPT)])
    pltpu.sync_copy(rinp, rin_o.at[pl.ds(r0, RPT)])
    def cv(v, _):
      sl = pl.ds(v * 16, 16)
      accb[sl] = rinp[sl] * routp[sl]
      return 0
    lax.fori_loop(0, NV, cv, 0)
    pltpu.sync_copy(accb, cmid_o.at[pl.ds(r0, RPT)])

  _zero_rows(g2, CHUNK)
  for q in range(QC):
    blk = pl.ds(r0 + q * CHUNK, CHUNK)
    pltpu.sync_copy(xh.at[c].at[blk], g0)
    _scale_block(g0, routp, q * CHUNK)
    pltpu.sync_copy(g0, A.at[blk])
    pltpu.sync_copy(g2, B.at[blk])
  plsc.subcore_barrier()

  _edge_pass(A, B, srcg, dstg, s, srcsb, dstsb, gbufs, gsems, ssems,
             is_s, is_d)
  plsc.subcore_barrier()
  for q in range(QC):
    blk = pl.ds(r0 + q * CHUNK, CHUNK)
    pltpu.sync_copy(B.at[blk], g0)
    _scale_block(g0, rinp, q * CHUNK)
    pltpu.sync_copy(g0, aggh.at[c].at[blk])


def _sc_hops_body(h1h, srcg, dstg, rin_i, rout_i, cmid_i, hh,
                  A, B, srcsb, dstsb, g0, g1, g2, g3,
                  routp, rinp, cmp_,
                  gs0, gs1, gs2, gs3, ss0, ss1, ss2, ss3, is_s, is_d):
  c = lax.axis_index("c")
  s = lax.axis_index("s")
  r0 = s * RPT
  gbufs = (g0, g1, g2, g3)
  gsems = (gs0, gs1, gs2, gs3)
  ssems = (ss0, ss1, ss2, ss3)

  pltpu.sync_copy(rout_i.at[pl.ds(r0, RPT)], routp)
  pltpu.sync_copy(rin_i.at[pl.ds(r0, RPT)], rinp)
  pltpu.sync_copy(cmid_i.at[pl.ds(r0, RPT)], cmp_)

  _zero_rows(g2, CHUNK)
  for q in range(QC):
    blk = pl.ds(r0 + q * CHUNK, CHUNK)
    pltpu.sync_copy(h1h.at[c].at[blk], g0)
    _scale_block(g0, routp, q * CHUNK)
    pltpu.sync_copy(g0, A.at[blk])
    pltpu.sync_copy(g2, B.at[blk])
  plsc.subcore_barrier()

  def hop(k, _):
    _edge_pass(A, B, srcg, dstg, s, srcsb, dstsb, gbufs, gsems, ssems,
               is_s, is_d)
    plsc.subcore_barrier()
    _zero_rows(g2, CHUNK)
    for q in range(QC):
      blk = pl.ds(r0 + q * CHUNK, CHUNK)
      pltpu.sync_copy(B.at[blk], g0)
      _scale_block(g0, cmp_, q * CHUNK)
      pltpu.sync_copy(g0, A.at[blk])
      pltpu.sync_copy(g2, B.at[blk])
    plsc.subcore_barrier()
    return 0
  lax.fori_loop(0, NHOP - 1, hop, 0)

  _edge_pass(A, B, srcg, dstg, s, srcsb, dstsb, gbufs, gsems, ssems,
             is_s, is_d)
  plsc.subcore_barrier()
  for q in range(QC):
    blk = pl.ds(r0 + q * CHUNK, CHUNK)
    pltpu.sync_copy(B.at[blk], g0)
    _scale_block(g0, rinp, q * CHUNK)
    pltpu.sync_copy(h1h.at[c].at[blk], g1)
    def addr(r, _):
      for j in range(HD // 16):
        sl = pl.ds(j * 16, 16)
        g0[r, sl] = g0[r, sl] + g1[r, sl]
      return 0
    lax.fori_loop(0, CHUNK, addr, 0)
    pltpu.sync_copy(g0, hh.at[c].at[blk])


_DMA = pltpu.SemaphoreType.DMA

_sc_stage1 = functools.partial(
    pl.kernel,
    compiler_params=_sc_params,
    out_type=(
        jax.ShapeDtypeStruct((NC, NPAD, HD), jnp.float32),
        jax.ShapeDtypeStruct((NPAD,), jnp.float32),
        jax.ShapeDtypeStruct((NPAD,), jnp.float32),
        jax.ShapeDtypeStruct((NPAD,), jnp.float32),
    ),
    mesh=_mesh,
    scratch_types=[
        pltpu.VMEM_SHARED((NPAD, HD), jnp.float32),
        pltpu.VMEM_SHARED((NPAD, HD), jnp.float32),
        pltpu.VMEM_SHARED((NPAD,), jnp.float32),
        pltpu.VMEM_SHARED((NPAD,), jnp.float32),
        pltpu.VMEM((2, SUP, CHUNK), jnp.int32),
        pltpu.VMEM((2, SUP, CHUNK), jnp.int32),
        pltpu.VMEM((CHUNK, HD), jnp.float32),
        pltpu.VMEM((CHUNK, HD), jnp.float32),
        pltpu.VMEM((CHUNK, HD), jnp.float32),
        pltpu.VMEM((CHUNK, HD), jnp.float32),
        pltpu.VMEM((CHUNK,), jnp.float32),
        pltpu.VMEM((RPT,), jnp.float32),
        pltpu.VMEM((RPT,), jnp.float32),
        pltpu.VMEM((RPT,), jnp.float32),
        pltpu.VMEM((RPT,), jnp.float32),
        _DMA, _DMA, _DMA, _DMA, _DMA, _DMA, _DMA, _DMA, _DMA, _DMA,
    ])(_sc_stage1_body)

_sc_hops = functools.partial(
    pl.kernel,
    compiler_params=_sc_params,
    out_type=jax.ShapeDtypeStruct((NC, NPAD, HD), jnp.float32),
    mesh=_mesh,
    scratch_types=[
        pltpu.VMEM_SHARED((NPAD, HD), jnp.float32),
        pltpu.VMEM_SHARED((NPAD, HD), jnp.float32),
        pltpu.VMEM((2, SUP, CHUNK), jnp.int32),
        pltpu.VMEM((2, SUP, CHUNK), jnp.int32),
        pltpu.VMEM((CHUNK, HD), jnp.float32),
        pltpu.VMEM((CHUNK, HD), jnp.float32),
        pltpu.VMEM((CHUNK, HD), jnp.float32),
        pltpu.VMEM((CHUNK, HD), jnp.float32),
        pltpu.VMEM((RPT,), jnp.float32),
        pltpu.VMEM((RPT,), jnp.float32),
        pltpu.VMEM((RPT,), jnp.float32),
        _DMA, _DMA, _DMA, _DMA, _DMA, _DMA, _DMA, _DMA, _DMA, _DMA,
    ])(_sc_hops_body)


BLK = 1024


def _tc_enc_body(aL, aR, We, be, Wt, bt, h1h, ht):
  lo, hi = pl.ds(0, HD), pl.ds(HD, HD)
  pre = aL[...] @ We[lo, :] + aR[...] @ We[hi, :] + be[...]
  h1 = jnp.maximum(pre, 0.0)
  h1h[0, :, :] = h1[:, :HD]
  h1h[1, :, :] = h1[:, HD:]
  pre_t = aL[...] @ Wt[lo, :] + aR[...] @ Wt[hi, :] + bt[...]
  ht[...] = jnp.maximum(pre_t, 0.0)


def _tc_pred_body(hL, hR, W1, b1, ap, W2, b2, out):
  lo, hi = pl.ds(0, HD), pl.ds(HD, HD)
  t = hL[...] @ W1[lo, :] + hR[...] @ W1[hi, :] + b1[...]
  t = jnp.where(t >= 0.0, t, ap[...] * t)
  out[...] = t @ W2[...] + b2[...]


def _full(shape):
  return pl.BlockSpec(shape, lambda i: tuple(0 for _ in shape))


def _tc_enc(aggh, W_enc, b_enc, W_tgt, b_tgt):
  return pl.pallas_call(
      _tc_enc_body,
      grid=(NPAD // BLK,),
      in_specs=[
          pl.BlockSpec((BLK, HD), lambda i: (i, 0)),
          pl.BlockSpec((BLK, HD), lambda i: (i, 0)),
          _full((D, D)), _full((1, D)), _full((D, D)), _full((1, D)),
      ],
      out_specs=[
          pl.BlockSpec((NC, BLK, HD), lambda i: (0, i, 0)),
          pl.BlockSpec((BLK, D), lambda i: (i, 0)),
      ],
      out_shape=[
          jax.ShapeDtypeStruct((NC, NPAD, HD), jnp.float32),
          jax.ShapeDtypeStruct((NPAD, D), jnp.float32),
      ],
  )(aggh[0], aggh[1], W_enc, b_enc.reshape(1, D), W_tgt, b_tgt.reshape(1, D))


def _tc_pred(hh, W1, b1, prelu_a, W2, b2):
  return pl.pallas_call(
      _tc_pred_body,
      grid=(NPAD // BLK,),
      in_specs=[
          pl.BlockSpec((BLK, HD), lambda i: (i, 0)),
          pl.BlockSpec((BLK, HD), lambda i: (i, 0)),
          _full((D, D)), _full((1, D)), _full((1, 1)), _full((D, D)),
          _full((1, D)),
      ],
      out_specs=pl.BlockSpec((BLK, D), lambda i: (i, 0)),
      out_shape=jax.ShapeDtypeStruct((NPAD, D), jnp.float32),
  )(hh[0], hh[1], W1, b1.reshape(1, D), prelu_a.reshape(1, 1), W2,
    b2.reshape(1, D))


def kernel(x, edge_index, W_enc, b_enc, W_tgt, b_tgt, W1, b1, prelu_a, W2, b2):
  src = edge_index[0]
  dst = edge_index[1]
  srcg = jnp.pad(src, (0, EPAD - E), constant_values=N).reshape(
      NS, NSUP, SUP, CHUNK)
  dstg = jnp.pad(dst, (0, EPAD - E), constant_values=N).reshape(
      NS, NSUP, SUP, CHUNK)
  xh = jnp.pad(x, ((0, NPAD - N), (0, 0))).reshape(NPAD, NC, HD).transpose(
      1, 0, 2)

  aggh, rin, rout, cmid = _sc_stage1(xh, srcg, dstg)
  h1h, h_target = _tc_enc(aggh, W_enc, b_enc, W_tgt, b_tgt)
  hh = _sc_hops(h1h, srcg, dstg, rin, rout, cmid)
  h_pred = _tc_pred(hh, W1, b1, prelu_a, W2, b2)

  h = jnp.concatenate([hh[0, :N], hh[1, :N]], axis=1)
  return h, h_pred[:N], h_target[:N]
